# baseline (speedup 1.0000x reference)
import functools

import jax
import jax.numpy as jnp
from jax import lax
from jax.experimental import pallas as pl
from jax.experimental.pallas import tpu as pltpu
from jax.experimental.pallas import tpu_sc as plsc

NUM_CODES = 8192
DIM = 256
TOKENS = 8192
BM = 1024
CHUNK_B = TOKENS // BM
CHUNK_T = CHUNK_B * BM

_SC_INFO = plsc.get_sparse_core_info()
_NC, _NS = _SC_INFO.num_cores, _SC_INFO.num_subcores
_NW = _NC * _NS
_BPW = CHUNK_T // _NW


def _dist_argmin_kernel(z_ref, w_ref, idx_ref, dmin_ref, csq_ref):
    @pl.when(pl.program_id(0) == 0)
    def _():
        w = w_ref[...]
        csq_ref[0, :] = jnp.sum(w * w, axis=1)

    RB = 128
    CH = 128
    NCH = NUM_CODES // CH

    def _row_block(r, _):
        z_r = jnp.transpose(z_ref[0, :, pl.ds(r * RB, RB)], (1, 0))
        zsq = jnp.sum(z_r * z_r, axis=1, keepdims=True)
        mm2 = lax.dot_general(z_r + z_r, w_ref[...], (((1,), (1,)), ((), ())),
                              preferred_element_type=jnp.float32)
        m = (zsq + csq_ref[0, 0:CH][None, :]) - mm2[:, 0:CH]
        c = jnp.zeros((RB, CH), jnp.int32)
        for k in range(1, NCH):
            v = (zsq + csq_ref[0, k * CH:(k + 1) * CH][None, :]) - mm2[:, k * CH:(k + 1) * CH]
            c = jnp.where(v < m, k, c)
            m = jnp.minimum(v, m)
        jj = c * CH + lax.broadcasted_iota(jnp.int32, (RB, CH), 1)
        mn = jnp.min(m, axis=1, keepdims=True)
        idx = jnp.min(jnp.where(m == mn, jj, jnp.int32(2**30)), axis=1)
        idx_ref[0, 0, pl.ds(r * RB, RB)] = idx
        dmin_ref[0, 0, pl.ds(r * RB, RB)] = mn[:, 0]

    for r in range(BM // RB):
        _row_block(r, None)


def _dist_argmin(z_chunk, weight):
    return pl.pallas_call(
        _dist_argmin_kernel,
        grid=(CHUNK_B,),
        in_specs=[
            pl.BlockSpec((1, DIM, BM), lambda i: (i, 0, 0)),
            pl.BlockSpec((NUM_CODES, DIM), lambda i: (0, 0)),
        ],
        out_specs=[
            pl.BlockSpec((1, 1, BM), lambda i: (i, 0, 0)),
            pl.BlockSpec((1, 1, BM), lambda i: (i, 0, 0)),
        ],
        out_shape=[
            jax.ShapeDtypeStruct((CHUNK_B, 1, BM), jnp.int32),
            jax.ShapeDtypeStruct((CHUNK_B, 1, BM), jnp.float32),
        ],
        scratch_shapes=[pltpu.VMEM((1, NUM_CODES), jnp.float32)],
        compiler_params=pltpu.CompilerParams(
            dimension_semantics=("arbitrary",),
        ),
    )(z_chunk, weight)


@functools.partial(
    pl.kernel,
    mesh=plsc.VectorSubcoreMesh(core_axis_name="c", subcore_axis_name="s"),
    out_type=jax.ShapeDtypeStruct((CHUNK_T, DIM), jnp.float32),
    scratch_types=[
        pltpu.VMEM((_BPW,), jnp.int32),
        pltpu.VMEM((_BPW, DIM), jnp.float32),
        pltpu.SemaphoreType.DMA,
    ],
)
def _sc_gather(table_hbm, idx_hbm, out_hbm, idx_v, rows_v, sem):
    wid = lax.axis_index("s") * _NC + lax.axis_index("c")
    base = wid * _BPW
    pltpu.sync_copy(idx_hbm.at[pl.ds(base, _BPW)], idx_v)
    pltpu.async_copy(table_hbm.at[idx_v], rows_v, sem).wait()
    pltpu.sync_copy(rows_v, out_hbm.at[pl.ds(base, _BPW)])


def kernel(z, weight):
    b, d, h, w = z.shape
    z3 = z.reshape(b, d, h * w)
    idx3, dmin3 = _dist_argmin(z3, weight)
    zq = _sc_gather(weight, idx3.reshape(-1))
    commitment_loss = jnp.sum(dmin3) / jnp.float32(TOKENS * DIM)
    loss = 0.25 * commitment_loss
    q = jnp.transpose(zq.reshape(b, h * w, d), (0, 2, 1)).reshape(b, d, h, w)
    return (q, loss, commitment_loss)


# speedup vs baseline: 1.3344x; 1.3344x over previous
import functools

import jax
import jax.numpy as jnp
from jax import lax
from jax.experimental import pallas as pl
from jax.experimental.pallas import tpu as pltpu
from jax.experimental.pallas import tpu_sc as plsc

NUM_CODES = 8192
DIM = 256
TOKENS = 8192
BM = 1024
CHUNK_B = TOKENS // BM
CHUNK_T = CHUNK_B * BM

_SC_INFO = plsc.get_sparse_core_info()
_NC, _NS = _SC_INFO.num_cores, _SC_INFO.num_subcores
_NW = _NC * _NS
_BPW = CHUNK_T // _NW


def _dist_argmin_kernel(z_ref, w_ref, idx_ref, dmin_ref, csq_ref):
    @pl.when(pl.program_id(0) == 0)
    def _():
        w = w_ref[...]
        csq_ref[0, :] = jnp.sum(w * w, axis=1)

    z = jnp.transpose(z_ref[0], (1, 0))
    RB = 256
    CH = 128
    NCH = NUM_CODES // CH
    for r in range(BM // RB):
        z_r = z[r * RB:(r + 1) * RB, :]
        zsq = jnp.sum(z_r * z_r, axis=1, keepdims=True)
        mm2 = lax.dot_general(z_r + z_r, w_ref[...], (((1,), (1,)), ((), ())),
                              preferred_element_type=jnp.float32)
        m = (zsq + csq_ref[0, 0:CH][None, :]) - mm2[:, 0:CH]
        c = jnp.zeros((RB, CH), jnp.int32)
        for k in range(1, NCH):
            v = (zsq + csq_ref[0, k * CH:(k + 1) * CH][None, :]) - mm2[:, k * CH:(k + 1) * CH]
            c = jnp.where(v < m, k, c)
            m = jnp.minimum(v, m)
        jj = c * CH + lax.broadcasted_iota(jnp.int32, (RB, CH), 1)
        mn = jnp.min(m, axis=1, keepdims=True)
        idx = jnp.min(jnp.where(m == mn, jj, jnp.int32(2**30)), axis=1)
        idx_ref[0, 0, r * RB:(r + 1) * RB] = idx
        dmin_ref[0, 0, r * RB:(r + 1) * RB] = mn[:, 0]


def _dist_argmin(z_chunk, weight):
    return pl.pallas_call(
        _dist_argmin_kernel,
        grid=(CHUNK_B,),
        in_specs=[
            pl.BlockSpec((1, DIM, BM), lambda i: (i, 0, 0)),
            pl.BlockSpec((NUM_CODES, DIM), lambda i: (0, 0)),
        ],
        out_specs=[
            pl.BlockSpec((1, 1, BM), lambda i: (i, 0, 0)),
            pl.BlockSpec((1, 1, BM), lambda i: (i, 0, 0)),
        ],
        out_shape=[
            jax.ShapeDtypeStruct((CHUNK_B, 1, BM), jnp.int32),
            jax.ShapeDtypeStruct((CHUNK_B, 1, BM), jnp.float32),
        ],
        scratch_shapes=[pltpu.VMEM((1, NUM_CODES), jnp.float32)],
        compiler_params=pltpu.CompilerParams(
            dimension_semantics=("arbitrary",),
        ),
    )(z_chunk, weight)


@functools.partial(
    pl.kernel,
    mesh=plsc.VectorSubcoreMesh(core_axis_name="c", subcore_axis_name="s"),
    out_type=jax.ShapeDtypeStruct((CHUNK_T, DIM), jnp.float32),
    scratch_types=[
        pltpu.VMEM((_BPW,), jnp.int32),
        pltpu.VMEM((_BPW, DIM), jnp.float32),
        pltpu.SemaphoreType.DMA,
    ],
)
def _sc_gather(table_hbm, idx_hbm, out_hbm, idx_v, rows_v, sem):
    wid = lax.axis_index("s") * _NC + lax.axis_index("c")
    base = wid * _BPW
    pltpu.sync_copy(idx_hbm.at[pl.ds(base, _BPW)], idx_v)
    pltpu.async_copy(table_hbm.at[idx_v], rows_v, sem).wait()
    pltpu.sync_copy(rows_v, out_hbm.at[pl.ds(base, _BPW)])


def kernel(z, weight):
    b, d, h, w = z.shape
    z3 = z.reshape(b, d, h * w)
    idx3, dmin3 = _dist_argmin(z3, weight)
    zq = _sc_gather(weight, idx3.reshape(-1))
    commitment_loss = jnp.sum(dmin3) / jnp.float32(TOKENS * DIM)
    loss = 0.25 * commitment_loss
    q = jnp.transpose(zq.reshape(b, h * w, d), (0, 2, 1)).reshape(b, d, h, w)
    return (q, loss, commitment_loss)
